# baseline (speedup 1.0000x reference)
import jax
import jax.numpy as jnp
from jax import lax
from jax.experimental import pallas as pl
from jax.experimental.pallas import tpu as pltpu
from jax.experimental.pallas import tpu_sc as plsc

N_NODES = 10000
D_IN = 128
D_OUT = 128
N_EDGES = 160000
N_REL = 3

NC = 2
NS = 16
NW = NC * NS
B = 80
EPT = 5120
NB = EPT // B
E_PAD = NW * EPT
NPAD = 10240
STRIPE = NPAD // NS
CROWS = NPAD // 128


def _sc_segment_sum_one(x, zeros_hbm, src_rel, dst_rel):
    mesh = plsc.VectorSubcoreMesh(core_axis_name="c", subcore_axis_name="s")

    def body(x_hbm, z_hbm, src_hbm, dst_hbm, out_hbm, cnt_out_hbm,
             src_v, dst_v, rows0, rows1, cnt_v, id_v, accum, cnt_sp,
             sem0, sem1):
        c = lax.axis_index("c")
        s = lax.axis_index("s")
        w = c * NS + s
        row0 = s * STRIPE

        for k in range(CROWS // 16):
            id_v[pl.ds(k * 16, 16)] = (
                lax.iota(jnp.int32, 16) + jnp.int32(k * 16))

        def hist_batch(j):
            ones16 = jnp.ones((16,), jnp.float32)
            for kk in range(B // 16):
                d16 = dst_v[j, 0, pl.ds(kk * 16, 16)]
                plsc.addupdate_scatter(
                    cnt_v, [lax.shift_right_logical(d16, 7),
                            lax.bitwise_and(d16, 127)], ones16)

        pltpu.sync_copy(src_hbm.at[pl.ds(w * EPT, EPT)], src_v)
        pltpu.sync_copy(dst_hbm.at[pl.ds(w * NB, NB)], dst_v)
        pltpu.sync_copy(z_hbm, accum.at[pl.ds(row0, STRIPE)])

        @pl.when(s == 0)
        def _():
            pltpu.sync_copy(z_hbm.at[pl.ds(0, CROWS)], cnt_sp)

        pltpu.async_copy(x_hbm.at[src_v.at[pl.ds(0, B)]], rows0, sem0)
        pltpu.async_copy(x_hbm.at[src_v.at[pl.ds(B, B)]], rows1, sem1)

        def zrow(i, carry):
            for kk in range(8):
                cnt_v[i, pl.ds(kk * 16, 16)] = jnp.zeros((16,), jnp.float32)
            return carry
        lax.fori_loop(0, CROWS, zrow, 0)

        plsc.subcore_barrier()

        def pair(k, carry):
            j0 = 2 * k
            j1 = j0 + 1
            hist_batch(j0)
            pltpu.make_async_copy(
                x_hbm.at[src_v.at[pl.ds(0, B)]], rows0, sem0).wait()
            pltpu.sync_copy(rows0, accum.at[dst_v.at[j0, 0]], add=True)

            @pl.when(j0 + 2 < NB)
            def _():
                pltpu.async_copy(
                    x_hbm.at[src_v.at[pl.ds((j0 + 2) * B, B)]], rows0, sem0)

            hist_batch(j1)
            pltpu.make_async_copy(
                x_hbm.at[src_v.at[pl.ds(B, B)]], rows1, sem1).wait()
            pltpu.sync_copy(rows1, accum.at[dst_v.at[j1, 0]], add=True)

            @pl.when(j1 + 2 < NB)
            def _():
                pltpu.async_copy(
                    x_hbm.at[src_v.at[pl.ds((j1 + 2) * B, B)]], rows1, sem1)
            return carry

        lax.fori_loop(0, NB // 2, pair, 0)

        pltpu.sync_copy(cnt_v, cnt_sp.at[id_v], add=True)
        plsc.subcore_barrier()

        pltpu.sync_copy(
            accum.at[pl.ds(row0, STRIPE)],
            out_hbm.at[pl.ds(c * NPAD + row0, STRIPE)])

        @pl.when(s == 0)
        def _():
            pltpu.sync_copy(cnt_sp, cnt_out_hbm.at[pl.ds(c * CROWS, CROWS)])

    run = pl.kernel(
        body,
        out_type=(
            jax.ShapeDtypeStruct((NC * NPAD, D_IN), jnp.float32),
            jax.ShapeDtypeStruct((NC * CROWS, 128), jnp.float32),
        ),
        mesh=mesh,
        compiler_params=pltpu.CompilerParams(needs_layout_passes=False),
        scratch_types=[
            pltpu.VMEM((EPT,), jnp.int32),
            pltpu.VMEM((NB, 1, B), jnp.int32),
            pltpu.VMEM((B, D_IN), jnp.float32),
            pltpu.VMEM((B, D_IN), jnp.float32),
            pltpu.VMEM((CROWS, 128), jnp.float32),
            pltpu.VMEM((CROWS,), jnp.int32),
            pltpu.VMEM_SHARED((NPAD, D_IN), jnp.float32),
            pltpu.VMEM_SHARED((CROWS, 128), jnp.float32),
            pltpu.SemaphoreType.DMA,
            pltpu.SemaphoreType.DMA,
        ],
    )
    return run(x, zeros_hbm, src_rel, dst_rel)


BN = 2048
CBN = BN // 128


def _tc_combine_body(s_ref, c_ref, w_ref, b_ref, o_ref):
    ri = lax.broadcasted_iota(jnp.int32, (BN, CBN), 0) // 128
    ci = lax.broadcasted_iota(jnp.int32, (BN, CBN), 1)
    m1 = (ri == ci).astype(jnp.float32)
    li = lax.broadcasted_iota(jnp.int32, (BN, 128), 0) % 128
    lj = lax.broadcasted_iota(jnp.int32, (BN, 128), 1)
    lane_mask = (li == lj).astype(jnp.float32)
    ones_col = jnp.ones((128, 1), jnp.float32)

    feat = s_ref[0] + s_ref[1]
    sel = jnp.dot(m1, c_ref[...], preferred_element_type=jnp.float32)
    cnt = jnp.dot(sel * lane_mask, ones_col,
                  preferred_element_type=jnp.float32)
    denom = jnp.maximum(cnt, 1.0)
    m = jnp.dot(feat, w_ref[...],
                preferred_element_type=jnp.float32) / denom
    o_ref[...] = m + jnp.where(cnt > 0.0, b_ref[...], 0.0)


def _tc_combine_one(s3, c2, w, b2):
    return pl.pallas_call(
        _tc_combine_body,
        grid=(NPAD // BN,),
        in_specs=[
            pl.BlockSpec((NC, BN, D_IN), lambda i: (0, i, 0)),
            pl.BlockSpec((CBN, 128), lambda i: (i, 0)),
            pl.BlockSpec((D_IN, D_OUT), lambda i: (0, 0)),
            pl.BlockSpec((1, D_OUT), lambda i: (0, 0)),
        ],
        out_specs=pl.BlockSpec((BN, D_OUT), lambda i: (i, 0)),
        out_shape=jax.ShapeDtypeStruct((NPAD, D_OUT), jnp.float32),
    )(s3, c2, w, b2)


def _prep_edges(ei):
    npad = E_PAD - N_EDGES
    pad_src = jnp.arange(npad, dtype=jnp.int32) % N_NODES
    pad_dst = N_NODES + (jnp.arange(npad, dtype=jnp.int32) % (NPAD - N_NODES))
    src = jnp.concatenate([ei[0], pad_src])
    dst = jnp.concatenate([ei[1], pad_dst]).reshape(NW * NB, 1, B)
    return src, dst


def kernel(x, W0, b0, W1, b1, W2, b2, edge_index0, edge_index1, edge_index2):
    zeros_hbm = jnp.zeros((STRIPE, D_IN), jnp.float32)

    out = None
    for w, b, ei in ((W0, b0, edge_index0), (W1, b1, edge_index1),
                     (W2, b2, edge_index2)):
        src, dst = _prep_edges(ei)
        s, cnt = _sc_segment_sum_one(x, zeros_hbm, src, dst)
        s3 = s.reshape(NC, NPAD, D_IN)
        cr = cnt.reshape(NC, CROWS, 128)
        m = _tc_combine_one(s3, cr[0] + cr[1], w, b.reshape(1, D_OUT))
        out = m if out is None else out + m
    return out[:N_NODES]


# speedup vs baseline: 1.1290x; 1.1290x over previous
import jax
import jax.numpy as jnp
from jax import lax
from jax.experimental import pallas as pl
from jax.experimental.pallas import tpu as pltpu
from jax.experimental.pallas import tpu_sc as plsc

N_NODES = 10000
D_IN = 128
D_OUT = 128
N_EDGES = 160000
N_REL = 3

NC = 2
NS = 16
NW = NC * NS
B = 80
EPT = 5120
NB = EPT // B
E_PAD = NW * EPT
NPAD = 10240
STRIPE = NPAD // NS
CROWS = NPAD // 128


def _sc_segment_sums(x, zeros_hbm, src_all, dst_all):
    mesh = plsc.VectorSubcoreMesh(core_axis_name="c", subcore_axis_name="s")

    def body(x_hbm, z_hbm, src_hbm, dst_hbm, out_hbm, cnt_out_hbm,
             src_v, dst_v, rows0, rows1, cnt_v, id_v, accum, cnt_sp,
             sem0, sem1, semz, semd):
        c = lax.axis_index("c")
        s = lax.axis_index("s")
        w = c * NS + s
        row0 = s * STRIPE

        for k in range(CROWS // 16):
            id_v[pl.ds(k * 16, 16)] = (
                lax.iota(jnp.int32, 16) + jnp.int32(k * 16))

        def hist_batch(j):
            ones16 = jnp.ones((16,), jnp.float32)
            for kk in range(B // 16):
                d16 = dst_v[j, 0, pl.ds(kk * 16, 16)]
                plsc.addupdate_scatter(
                    cnt_v, [lax.shift_right_logical(d16, 7),
                            lax.bitwise_and(d16, 127)], ones16)

        for r in range(N_REL):
            base = r * NW + w
            pltpu.async_copy(src_hbm.at[pl.ds(base * EPT, EPT)], src_v,
                             sem0)
            pltpu.async_copy(dst_hbm.at[pl.ds(base * NB, NB)], dst_v, sem1)
            if r > 0:
                pltpu.make_async_copy(
                    accum.at[pl.ds(row0, STRIPE)],
                    out_hbm.at[pl.ds(((r - 1) * NC + c) * NPAD + row0,
                                     STRIPE)],
                    semd).wait()
            pltpu.async_copy(z_hbm, accum.at[pl.ds(row0, STRIPE)], semz)

            @pl.when(s == 0)
            def _():
                pltpu.sync_copy(z_hbm.at[pl.ds(0, CROWS)], cnt_sp)

            def zrow(i, carry):
                for kk in range(8):
                    cnt_v[i, pl.ds(kk * 16, 16)] = jnp.zeros((16,),
                                                             jnp.float32)
                return carry
            lax.fori_loop(0, CROWS, zrow, 0)

            pltpu.make_async_copy(src_hbm.at[pl.ds(base * EPT, EPT)],
                                  src_v, sem0).wait()
            pltpu.async_copy(x_hbm.at[src_v.at[pl.ds(0, B)]], rows0, sem0)
            pltpu.make_async_copy(dst_hbm.at[pl.ds(base * NB, NB)], dst_v,
                                  sem1).wait()
            pltpu.async_copy(x_hbm.at[src_v.at[pl.ds(B, B)]], rows1, sem1)
            pltpu.make_async_copy(z_hbm, accum.at[pl.ds(row0, STRIPE)],
                                  semz).wait()
            plsc.subcore_barrier()

            def pair(k, carry):
                j0 = 2 * k
                j1 = j0 + 1
                hist_batch(j0)
                pltpu.make_async_copy(
                    x_hbm.at[src_v.at[pl.ds(0, B)]], rows0, sem0).wait()
                pltpu.sync_copy(rows0, accum.at[dst_v.at[j0, 0]], add=True)

                @pl.when(j0 + 2 < NB)
                def _():
                    pltpu.async_copy(
                        x_hbm.at[src_v.at[pl.ds((j0 + 2) * B, B)]],
                        rows0, sem0)

                hist_batch(j1)
                pltpu.make_async_copy(
                    x_hbm.at[src_v.at[pl.ds(B, B)]], rows1, sem1).wait()
                pltpu.sync_copy(rows1, accum.at[dst_v.at[j1, 0]], add=True)

                @pl.when(j1 + 2 < NB)
                def _():
                    pltpu.async_copy(
                        x_hbm.at[src_v.at[pl.ds((j1 + 2) * B, B)]],
                        rows1, sem1)
                return carry

            lax.fori_loop(0, NB // 2, pair, 0)

            pltpu.sync_copy(cnt_v, cnt_sp.at[id_v], add=True)
            plsc.subcore_barrier()

            pltpu.async_copy(
                accum.at[pl.ds(row0, STRIPE)],
                out_hbm.at[pl.ds((r * NC + c) * NPAD + row0, STRIPE)],
                semd)

            @pl.when(s == 0)
            def _():
                pltpu.sync_copy(
                    cnt_sp,
                    cnt_out_hbm.at[pl.ds((r * NC + c) * CROWS, CROWS)])

        pltpu.make_async_copy(
            accum.at[pl.ds(row0, STRIPE)],
            out_hbm.at[pl.ds(((N_REL - 1) * NC + c) * NPAD + row0, STRIPE)],
            semd).wait()

    run = pl.kernel(
        body,
        out_type=(
            jax.ShapeDtypeStruct((N_REL * NC * NPAD, D_IN), jnp.float32),
            jax.ShapeDtypeStruct((N_REL * NC * CROWS, 128), jnp.float32),
        ),
        mesh=mesh,
        compiler_params=pltpu.CompilerParams(needs_layout_passes=False),
        scratch_types=[
            pltpu.VMEM((EPT,), jnp.int32),
            pltpu.VMEM((NB, 1, B), jnp.int32),
            pltpu.VMEM((B, D_IN), jnp.float32),
            pltpu.VMEM((B, D_IN), jnp.float32),
            pltpu.VMEM((CROWS, 128), jnp.float32),
            pltpu.VMEM((CROWS,), jnp.int32),
            pltpu.VMEM_SHARED((NPAD, D_IN), jnp.float32),
            pltpu.VMEM_SHARED((CROWS, 128), jnp.float32),
            pltpu.SemaphoreType.DMA,
            pltpu.SemaphoreType.DMA,
            pltpu.SemaphoreType.DMA,
            pltpu.SemaphoreType.DMA,
        ],
    )
    return run(x, zeros_hbm, src_all, dst_all)


BN = 2048
CBN = BN // 128


def _tc_combine_body(s_ref, c_ref, w_ref, b_ref, o_ref):
    ri = lax.broadcasted_iota(jnp.int32, (BN, CBN), 0) // 128
    ci = lax.broadcasted_iota(jnp.int32, (BN, CBN), 1)
    m1 = (ri == ci).astype(jnp.float32)
    li = lax.broadcasted_iota(jnp.int32, (BN, 128), 0) % 128
    lj = lax.broadcasted_iota(jnp.int32, (BN, 128), 1)
    lane_mask = (li == lj).astype(jnp.float32)
    ones_col = jnp.ones((128, 1), jnp.float32)

    acc = jnp.zeros((BN, D_OUT), jnp.float32)
    for r in range(N_REL):
        feat = s_ref[r, 0] + s_ref[r, 1]
        sel = jnp.dot(m1, c_ref[r],
                      preferred_element_type=jnp.float32)
        cnt = jnp.dot(sel * lane_mask, ones_col,
                      preferred_element_type=jnp.float32)
        denom = jnp.maximum(cnt, 1.0)
        m = jnp.dot(feat, w_ref[r],
                    preferred_element_type=jnp.float32) / denom
        m = m + jnp.where(cnt > 0.0, b_ref[r], 0.0)
        acc = acc + m
    o_ref[...] = acc


def _tc_combine(s4, c3, ws, bs):
    return pl.pallas_call(
        _tc_combine_body,
        grid=(NPAD // BN,),
        in_specs=[
            pl.BlockSpec((N_REL, NC, BN, D_IN), lambda i: (0, 0, i, 0)),
            pl.BlockSpec((N_REL, CBN, 128), lambda i: (0, i, 0)),
            pl.BlockSpec((N_REL, D_IN, D_OUT), lambda i: (0, 0, 0)),
            pl.BlockSpec((N_REL, 1, D_OUT), lambda i: (0, 0, 0)),
        ],
        out_specs=pl.BlockSpec((BN, D_OUT), lambda i: (i, 0)),
        out_shape=jax.ShapeDtypeStruct((NPAD, D_OUT), jnp.float32),
    )(s4, c3, ws, bs)


def kernel(x, W0, b0, W1, b1, W2, b2, edge_index0, edge_index1, edge_index2):
    zeros_hbm = jnp.zeros((STRIPE, D_IN), jnp.float32)

    npad = E_PAD - N_EDGES
    pad_src = jnp.arange(npad, dtype=jnp.int32) % N_NODES
    pad_dst = N_NODES + (jnp.arange(npad, dtype=jnp.int32) % (NPAD - N_NODES))
    eis = (edge_index0, edge_index1, edge_index2)
    src_all = jnp.concatenate(
        [a for ei in eis for a in (ei[0], pad_src)])
    dst_all = jnp.concatenate(
        [a for ei in eis for a in (ei[1], pad_dst)]
    ).reshape(N_REL * NW * NB, 1, B)

    s, cnt = _sc_segment_sums(x, zeros_hbm, src_all, dst_all)
    s4 = s.reshape(N_REL, NC, NPAD, D_IN)
    cr = cnt.reshape(N_REL, NC, CROWS, 128)
    c3 = cr[:, 0] + cr[:, 1]

    ws = jnp.stack([W0, W1, W2])
    bs = jnp.stack([b0, b1, b2]).reshape(N_REL, 1, D_OUT)
    return _tc_combine(s4, c3, ws, bs)[:N_NODES]
